# baseline (device time: 41223 ns/iter reference)
import jax
import jax.numpy as jnp
from jax import lax
from jax.experimental import pallas as pl
from jax.experimental.pallas import tpu as pltpu

N_DEV = 4
B = 64
D = 1024
H = 2048
BF = jnp.bfloat16
F32 = jnp.float32


def kernel(x, Win0, Wout0, Win1, Wout1, Win2, Wout2):
    def body(x_ref, win0_ref, wout0_ref, win1_ref, wout1_ref, win2_ref,
             wout2_ref, out_ref,
             xacts, sendbuf, rsbuf, winbuf, woutbuf, winb, woutb,
             ag_send, ag_recv, rs_send, rs_recv, wsem):
        my = lax.axis_index("i")

        w_refs = ((win0_ref, wout0_ref), (win1_ref, wout1_ref),
                  (win2_ref, wout2_ref))

        def start_wcopy(l):
            ci = pltpu.make_async_copy(w_refs[l][0], winbuf.at[l % 2],
                                       wsem.at[2 * l])
            co = pltpu.make_async_copy(w_refs[l][1], woutbuf.at[l % 2],
                                       wsem.at[2 * l + 1])
            ci.start()
            co.start()
            return (ci, co)

        wcopies = [start_wcopy(0), start_wcopy(1), None]

        xacts[my] = x_ref[...].astype(BF)

        barrier = pltpu.get_barrier_semaphore()
        for t in (1, 2, 3):
            pl.semaphore_signal(barrier, inc=1,
                                device_id=((my + t) % N_DEV,),
                                device_id_type=pl.DeviceIdType.MESH)
        pl.semaphore_wait(barrier, 3)

        def ag_descriptor(t):
            return pltpu.make_async_remote_copy(
                src_ref=xacts.at[my],
                dst_ref=xacts.at[my],
                send_sem=ag_send.at[t],
                recv_sem=ag_recv.at[t],
                device_id=((my + t) % N_DEV,),
                device_id_type=pl.DeviceIdType.MESH,
            )

        def ag_send_all():
            for t in (1, 2, 3):
                ag_descriptor(t).start()

        def ag_wait_send_all():
            for t in (1, 2, 3):
                ag_descriptor(t).wait_send()

        def wait_ag_half(half):
            for t in (1, 2, 3):
                p = (my + t) % N_DEV

                @pl.when((p // 2) == half)
                def _(t=t, p=p):
                    pltpu.make_async_remote_copy(
                        src_ref=xacts.at[my],
                        dst_ref=xacts.at[p],
                        send_sem=ag_send.at[t],
                        recv_sem=ag_recv.at[4 - t],
                        device_id=(0,),
                        device_id_type=pl.DeviceIdType.MESH,
                    ).wait_recv()

        def half_compute(half):
            base = 2 * half
            xa = xacts[base:base + 2].reshape(2 * B, D)
            h = jnp.maximum(
                jnp.dot(xa, winb[...], preferred_element_type=F32),
                0.0).astype(BF)
            part = jnp.dot(h, woutb[...], preferred_element_type=F32)
            sendbuf[base:base + 2] = part.reshape(2, B, D).astype(BF)
            for t in (1, 2, 3):
                p = (my + t) % N_DEV

                @pl.when((p // 2) == half)
                def _(t=t, p=p):
                    pltpu.make_async_remote_copy(
                        src_ref=sendbuf.at[p],
                        dst_ref=rsbuf.at[t],
                        send_sem=rs_send.at[t],
                        recv_sem=rs_recv.at[t],
                        device_id=(p,),
                        device_id_type=pl.DeviceIdType.MESH,
                    ).start()

        def rs_descriptor(t):
            return pltpu.make_async_remote_copy(
                src_ref=sendbuf.at[my],
                dst_ref=rsbuf.at[4 - t],
                send_sem=rs_send.at[t],
                recv_sem=rs_recv.at[4 - t],
                device_id=(0,),
                device_id_type=pl.DeviceIdType.MESH,
            )

        def my_block_finish(l):
            for t in (1, 2, 3):
                rs_descriptor(t).wait_recv()
            acc = (sendbuf[my].astype(F32) + rsbuf[1].astype(F32)
                   + rsbuf[2].astype(F32) + rsbuf[3].astype(F32))
            if l == 2:
                out_ref[...] = acc
            else:
                ag_wait_send_all()
                xacts[my] = acc.astype(BF)
                ag_send_all()

        def cast_win(l):
            wcopies[l][0].wait()
            winb[...] = winbuf[l % 2].astype(BF)

        def cast_wout(l):
            wcopies[l][1].wait()
            woutb[...] = woutbuf[l % 2].astype(BF)

        ag_send_all()

        cast_win(0)
        cast_wout(0)
        wcopies[2] = start_wcopy(2)

        for l in (0, 1, 2):
            wait_ag_half(0)
            half_compute(0)
            wait_ag_half(1)
            half_compute(1)
            if l < 2:
                cast_win(l + 1)
            my_block_finish(l)
            if l < 2:
                cast_wout(l + 1)

            for t in (1, 2, 3):
                pltpu.make_async_remote_copy(
                    src_ref=sendbuf.at[(my + t) % N_DEV],
                    dst_ref=rsbuf.at[t],
                    send_sem=rs_send.at[t],
                    recv_sem=rs_recv.at[t],
                    device_id=(0,),
                    device_id_type=pl.DeviceIdType.MESH,
                ).wait_send()

        ag_wait_send_all()

    return pl.pallas_call(
        body,
        out_shape=jax.ShapeDtypeStruct((B, D), jnp.float32),
        in_specs=[pl.BlockSpec(memory_space=pltpu.VMEM)]
        + [pl.BlockSpec(memory_space=pl.ANY)] * 6,
        out_specs=pl.BlockSpec(memory_space=pltpu.VMEM),
        scratch_shapes=[
            pltpu.VMEM((N_DEV, B, D), BF),
            pltpu.VMEM((N_DEV, B, D), BF),
            pltpu.VMEM((N_DEV, B, D), BF),
            pltpu.VMEM((2, D, H), F32),
            pltpu.VMEM((2, H, D), F32),
            pltpu.VMEM((D, H), BF),
            pltpu.VMEM((H, D), BF),
            pltpu.SemaphoreType.DMA((N_DEV,)),
            pltpu.SemaphoreType.DMA((N_DEV,)),
            pltpu.SemaphoreType.DMA((N_DEV,)),
            pltpu.SemaphoreType.DMA((N_DEV,)),
            pltpu.SemaphoreType.DMA((6,)),
        ],
        compiler_params=pltpu.CompilerParams(
            collective_id=0, vmem_limit_bytes=100 * 1024 * 1024),
    )(x, Win0, Wout0, Win1, Wout1, Win2, Wout2)


# device time: 41069 ns/iter; 1.0037x vs baseline; 1.0037x over previous
import jax
import jax.numpy as jnp
from jax import lax
from jax.experimental import pallas as pl
from jax.experimental.pallas import tpu as pltpu

N_DEV = 4
B = 64
D = 1024
H = 2048
BF = jnp.bfloat16
F32 = jnp.float32


def kernel(x, Win0, Wout0, Win1, Wout1, Win2, Wout2):
    def body(x_ref, win0_ref, wout0_ref, win1_ref, wout1_ref, win2_ref,
             wout2_ref, out_ref,
             xacts, sendbuf, rsbuf, winbuf, woutbuf, winb, woutb,
             ag_send, ag_recv, rs_send, rs_recv, wsem):
        my = lax.axis_index("i")

        w_refs = ((win0_ref, wout0_ref), (win1_ref, wout1_ref),
                  (win2_ref, wout2_ref))

        def start_wcopy(l):
            ci = pltpu.make_async_copy(w_refs[l][0], winbuf.at[l % 2],
                                       wsem.at[2 * l])
            co = pltpu.make_async_copy(w_refs[l][1], woutbuf.at[l % 2],
                                       wsem.at[2 * l + 1])
            ci.start()
            co.start()
            return (ci, co)

        wcopies = [start_wcopy(0), None, None]

        xacts[my] = x_ref[...].astype(BF)

        barrier = pltpu.get_barrier_semaphore()
        for t in (1, 2, 3):
            pl.semaphore_signal(barrier, inc=1,
                                device_id=((my + t) % N_DEV,),
                                device_id_type=pl.DeviceIdType.MESH)
        pl.semaphore_wait(barrier, 3)

        def ag_descriptor(t):
            return pltpu.make_async_remote_copy(
                src_ref=xacts.at[my],
                dst_ref=xacts.at[my],
                send_sem=ag_send.at[t],
                recv_sem=ag_recv.at[t],
                device_id=((my + t) % N_DEV,),
                device_id_type=pl.DeviceIdType.MESH,
            )

        def ag_send_all():
            for t in (1, 2, 3):
                ag_descriptor(t).start()

        def ag_wait_send_all():
            for t in (1, 2, 3):
                ag_descriptor(t).wait_send()

        def wait_ag_half(half):
            for t in (1, 2, 3):
                p = (my + t) % N_DEV

                @pl.when((p // 2) == half)
                def _(t=t, p=p):
                    pltpu.make_async_remote_copy(
                        src_ref=xacts.at[my],
                        dst_ref=xacts.at[p],
                        send_sem=ag_send.at[t],
                        recv_sem=ag_recv.at[4 - t],
                        device_id=(0,),
                        device_id_type=pl.DeviceIdType.MESH,
                    ).wait_recv()

        def half_compute(half):
            base = 2 * half
            xa = xacts[base:base + 2].reshape(2 * B, D)
            h = jnp.maximum(
                jnp.dot(xa, winb[...], preferred_element_type=F32),
                0.0).astype(BF)
            part = jnp.dot(h, woutb[...], preferred_element_type=F32)
            sendbuf[base:base + 2] = part.reshape(2, B, D).astype(BF)
            for t in (1, 2, 3):
                p = (my + t) % N_DEV

                @pl.when((p // 2) == half)
                def _(t=t, p=p):
                    pltpu.make_async_remote_copy(
                        src_ref=sendbuf.at[p],
                        dst_ref=rsbuf.at[t],
                        send_sem=rs_send.at[t],
                        recv_sem=rs_recv.at[t],
                        device_id=(p,),
                        device_id_type=pl.DeviceIdType.MESH,
                    ).start()

        def rs_descriptor(t):
            return pltpu.make_async_remote_copy(
                src_ref=sendbuf.at[my],
                dst_ref=rsbuf.at[4 - t],
                send_sem=rs_send.at[t],
                recv_sem=rs_recv.at[4 - t],
                device_id=(0,),
                device_id_type=pl.DeviceIdType.MESH,
            )

        def my_block_finish(l):
            for t in (1, 2, 3):
                rs_descriptor(t).wait_recv()
            acc = (sendbuf[my].astype(F32) + rsbuf[1].astype(F32)
                   + rsbuf[2].astype(F32) + rsbuf[3].astype(F32))
            if l == 2:
                out_ref[...] = acc
            else:
                ag_wait_send_all()
                xacts[my] = acc.astype(BF)
                ag_send_all()

        def cast_win(l):
            wcopies[l][0].wait()
            winb[...] = winbuf[l % 2].astype(BF)

        def cast_wout(l):
            wcopies[l][1].wait()
            woutb[...] = woutbuf[l % 2].astype(BF)

        ag_send_all()

        cast_win(0)
        cast_wout(0)
        wcopies[1] = start_wcopy(1)
        wcopies[2] = start_wcopy(2)

        for l in (0, 1, 2):
            wait_ag_half(0)
            half_compute(0)
            wait_ag_half(1)
            half_compute(1)
            if l < 2:
                cast_win(l + 1)
            my_block_finish(l)
            if l < 2:
                cast_wout(l + 1)

            for t in (1, 2, 3):
                pltpu.make_async_remote_copy(
                    src_ref=sendbuf.at[(my + t) % N_DEV],
                    dst_ref=rsbuf.at[t],
                    send_sem=rs_send.at[t],
                    recv_sem=rs_recv.at[t],
                    device_id=(0,),
                    device_id_type=pl.DeviceIdType.MESH,
                ).wait_send()

        ag_wait_send_all()

    return pl.pallas_call(
        body,
        out_shape=jax.ShapeDtypeStruct((B, D), jnp.float32),
        in_specs=[pl.BlockSpec(memory_space=pltpu.VMEM)]
        + [pl.BlockSpec(memory_space=pl.ANY)] * 6,
        out_specs=pl.BlockSpec(memory_space=pltpu.VMEM),
        scratch_shapes=[
            pltpu.VMEM((N_DEV, B, D), BF),
            pltpu.VMEM((N_DEV, B, D), BF),
            pltpu.VMEM((N_DEV, B, D), BF),
            pltpu.VMEM((2, D, H), F32),
            pltpu.VMEM((2, H, D), F32),
            pltpu.VMEM((D, H), BF),
            pltpu.VMEM((H, D), BF),
            pltpu.SemaphoreType.DMA((N_DEV,)),
            pltpu.SemaphoreType.DMA((N_DEV,)),
            pltpu.SemaphoreType.DMA((N_DEV,)),
            pltpu.SemaphoreType.DMA((N_DEV,)),
            pltpu.SemaphoreType.DMA((6,)),
        ],
        compiler_params=pltpu.CompilerParams(
            collective_id=0, vmem_limit_bytes=100 * 1024 * 1024),
    )(x, Win0, Wout0, Win1, Wout1, Win2, Wout2)
